# baseline (speedup 1.0000x reference)
import functools

import jax
import jax.numpy as jnp
from jax import lax
from jax.experimental import pallas as pl
from jax.experimental.pallas import tpu as pltpu
from jax.experimental.pallas import tpu_sc as plsc

NUM_SEQ = 16384
SEQ = 50
DIM = 64
NC = 2
NS = 16
NW = NC * NS
BG = 128
NBG = NUM_SEQ // BG
BG_PER_W = NBG // NW
BLOCKS_PER_W = BG_PER_W * SEQ
NBUF = 3
L = 16

_mesh = plsc.VectorSubcoreMesh(core_axis_name="c", subcore_axis_name="s")


@functools.partial(
    pl.kernel,
    mesh=_mesh,
    out_type=jax.ShapeDtypeStruct((SEQ, DIM, NUM_SEQ), jnp.float32),
    scratch_types=[
        pltpu.VMEM((SEQ, BG_PER_W * BG), jnp.int32),
        pltpu.VMEM((NBUF, BG, 2 * DIM), jnp.float32),
        pltpu.VMEM((NBUF, DIM, BG), jnp.float32),
        pltpu.VMEM((NBUF, BG), jnp.int32),
        pltpu.SemaphoreType.DMA((NBUF,)),
        pltpu.SemaphoreType.DMA((NBUF,)),
    ],
    compiler_params=pltpu.CompilerParams(
        use_tc_tiling_on_sc=True, needs_layout_passes=False
    ),
)
def _embed_kernel(idsT_hbm, table_hbm, out_hbm, idx_v, blk_v, out_v,
                  pidx_v, sem_g, sem_w):
    wid = lax.axis_index("s") * NC + lax.axis_index("c")
    b_base = pl.multiple_of(wid * (BG_PER_W * BG), 128)
    pltpu.sync_copy(idsT_hbm.at[:, pl.ds(b_base, BG_PER_W * BG)], idx_v)

    lane = lax.iota(jnp.int32, L)

    def block_sg(n):
        s = lax.rem(n, SEQ)
        g = lax.div(n, SEQ)
        return s, g

    def prep_and_gather(slot, n):
        s, g = block_sg(n)
        off = g * BG
        for q in range(BG // L):
            v = idx_v[s, pl.ds(off + q * L, L)]
            pidx_v[slot, pl.ds(q * L, L)] = lax.shift_right_logical(v, 1)
        pltpu.make_async_copy(
            table_hbm.at[pidx_v.at[slot]], blk_v.at[slot], sem_g.at[slot]
        ).start()

    def wait_gather(slot):
        pltpu.make_async_copy(
            table_hbm.at[pidx_v.at[slot]], blk_v.at[slot], sem_g.at[slot]
        ).wait()

    def transpose_block(slot, n):
        s, g = block_sg(n)
        off = g * BG
        blk = blk_v.at[slot]
        for q in range(BG // L):
            v = idx_v[s, pl.ds(off + q * L, L)]
            cbase = lax.shift_left(lax.bitwise_and(v, 1), 6)
            rows = lane + (q * L)
            for d in range(DIM):
                vals = plsc.load_gather(blk, [rows, cbase + d])
                out_v[slot, d, pl.ds(q * L, L)] = vals

    def start_write(slot, n):
        s, g = block_sg(n)
        b0 = b_base + g * BG
        pltpu.make_async_copy(
            out_v.at[slot], out_hbm.at[s, :, pl.ds(b0, BG)], sem_w.at[slot]
        ).start()

    def wait_write(slot, n):
        s, g = block_sg(n)
        b0 = b_base + g * BG
        pltpu.make_async_copy(
            out_v.at[slot], out_hbm.at[s, :, pl.ds(b0, BG)], sem_w.at[slot]
        ).wait()

    for b in range(NBUF):
        prep_and_gather(b, b)

    def body(n, carry):
        def step(slot):
            wait_gather(slot)
            transpose_block(slot, n)
            start_write(slot, n)
            @pl.when(n + NBUF < BLOCKS_PER_W)
            def _():
                wait_write(slot, n)
                prep_and_gather(slot, n + NBUF)
            @pl.when(n + NBUF >= BLOCKS_PER_W)
            def _():
                wait_write(slot, n)
            return ()

        slot_id = lax.rem(n, NBUF)
        for b in range(NBUF):
            @pl.when(slot_id == b)
            def _():
                step(b)
        return carry

    lax.fori_loop(0, BLOCKS_PER_W, body, 0)


def kernel(input_ids, weight):
    idsT = input_ids.astype(jnp.int32).T
    tableP = weight.reshape(500000, 2 * DIM)
    out = _embed_kernel(idsT, tableP)
    return out.transpose(2, 0, 1)


# speedup vs baseline: 1.5641x; 1.5641x over previous
import functools

import jax
import jax.numpy as jnp
from jax import lax
from jax.experimental import pallas as pl
from jax.experimental.pallas import tpu as pltpu
from jax.experimental.pallas import tpu_sc as plsc

NUM_SEQ = 16384
SEQ = 50
DIM = 64
NC = 2
NS = 16
NW = NC * NS
CHUNK = 128
NBLK = NUM_SEQ * SEQ // CHUNK
BLOCKS_PER_W = NBLK // NW
G = 4
NGRP = BLOCKS_PER_W // G
NBUF = 2

_mesh = plsc.VectorSubcoreMesh(core_axis_name="c", subcore_axis_name="s")


@functools.partial(
    pl.kernel,
    mesh=_mesh,
    out_type=jax.ShapeDtypeStruct((NBLK, CHUNK, DIM), jnp.float32),
    scratch_types=[
        pltpu.VMEM((BLOCKS_PER_W, CHUNK), jnp.int32),
        pltpu.VMEM((NBUF, G, CHUNK, DIM), jnp.float32),
        pltpu.SemaphoreType.DMA((NBUF,)),
        pltpu.SemaphoreType.DMA((NBUF,)),
    ],
    compiler_params=pltpu.CompilerParams(use_tc_tiling_on_sc=False),
)
def _embed_kernel(ids_hbm, table_hbm, out_hbm, idx_v, rows_v, sem_g, sem_w):
    wid = lax.axis_index("s") * NC + lax.axis_index("c")
    r0 = wid * BLOCKS_PER_W
    pltpu.sync_copy(ids_hbm.at[pl.ds(r0, BLOCKS_PER_W)], idx_v)

    def gather_descs(buf, grp):
        return [
            pltpu.make_async_copy(
                table_hbm.at[idx_v.at[grp * G + j]],
                rows_v.at[buf, j],
                sem_g.at[buf],
            )
            for j in range(G)
        ]

    def write_desc(buf, grp):
        return pltpu.make_async_copy(
            rows_v.at[buf], out_hbm.at[pl.ds(r0 + grp * G, G)], sem_w.at[buf]
        )

    def start_gathers(buf, grp):
        for d in gather_descs(buf, grp):
            d.start()

    def step(buf, grp):
        for d in gather_descs(buf, grp):
            d.wait()
        write_desc(buf, grp).start()
        write_desc(buf, grp).wait()

        @pl.when(grp + NBUF < NGRP)
        def _():
            start_gathers(buf, grp + NBUF)

    for b in range(NBUF):
        start_gathers(b, b)

    def body(t, carry):
        for b in range(NBUF):
            step(b, t * NBUF + b)
        return carry

    lax.fori_loop(0, NGRP // NBUF, body, 0)


def kernel(input_ids, weight):
    ids = input_ids.astype(jnp.int32).reshape(NBLK, CHUNK)
    out = _embed_kernel(ids, weight)
    return out.reshape(NUM_SEQ, SEQ, DIM)
